# baseline (speedup 1.0000x reference)
import functools

import jax
import jax.numpy as jnp
import numpy as np
from jax import lax
from jax.experimental import pallas as pl
from jax.experimental.pallas import tpu as pltpu
from jax.experimental.pallas import tpu_sc as plsc

_ALPHA = 4


def _linspace_trunc_idx(t: int) -> tuple:
    with jax.ensure_compile_time_eval():
        v = jnp.linspace(0.0, t - 1, t // _ALPHA).astype(jnp.int32)
    return tuple(int(i) for i in np.asarray(v))


def _sc_gather(flat, C, T, H, W, n):
    nslab = C * n
    ROWS = 64
    per_slab = H // ROWS
    nunit = nslab * per_slab
    mesh = plsc.VectorSubcoreMesh(core_axis_name="c", subcore_axis_name="s")
    info = plsc.get_sparse_core_info()
    nworker = info.num_cores * info.num_subcores
    per_worker = nunit // nworker
    assert nunit % nworker == 0

    @functools.partial(
        pl.kernel,
        mesh=mesh,
        out_type=jax.ShapeDtypeStruct((nslab, H, W), flat.dtype),
        scratch_types=[
            pltpu.VMEM((per_worker, ROWS, W), flat.dtype),
            pltpu.SemaphoreType.DMA((per_worker,)),
            pltpu.SemaphoreType.DMA((per_worker,)),
        ],
    )
    def sc_kernel(x_hbm, o_hbm, buf, in_sem, out_sem):
        wid = lax.axis_index("s") * info.num_cores + lax.axis_index("c")

        def slices(k):
            u = wid * per_worker + k
            j = u // per_slab
            p = u % per_slab
            c = j // n
            t = j % n
            src = c * T + t * (T - 1) // (n - 1)
            row0 = p * ROWS
            return (
                x_hbm.at[src, pl.ds(row0, ROWS), :],
                o_hbm.at[j, pl.ds(row0, ROWS), :],
            )

        ins = []
        for k in range(per_worker):
            src_slice, _ = slices(k)
            ins.append(pltpu.async_copy(src_slice, buf.at[k], in_sem.at[k]))
        outs = []
        for k in range(per_worker):
            _, dst_slice = slices(k)
            ins[k].wait()
            outs.append(pltpu.async_copy(buf.at[k], dst_slice, out_sem.at[k]))
        for cp in outs:
            cp.wait()

    return sc_kernel(flat)


def _tc_copy(flat):
    N, H, W = flat.shape
    CH = 16
    nchunk = N // CH
    DEPTH = 3

    def body(src, dst, buf, in_sem, out_sem):
        def start_in(ch):
            b = ch % DEPTH
            pltpu.make_async_copy(
                src.at[pl.ds(ch * CH, CH)], buf.at[b], in_sem.at[b]
            ).start()

        def wait_in(ch):
            b = ch % DEPTH
            pltpu.make_async_copy(
                src.at[pl.ds(ch * CH, CH)], buf.at[b], in_sem.at[b]
            ).wait()

        def start_out(ch):
            b = ch % DEPTH
            pltpu.make_async_copy(
                buf.at[b], dst.at[pl.ds(ch * CH, CH)], out_sem.at[b]
            ).start()

        def wait_out(ch):
            b = ch % DEPTH
            pltpu.make_async_copy(
                buf.at[b], dst.at[pl.ds(ch * CH, CH)], out_sem.at[b]
            ).wait()

        for ch in range(min(DEPTH - 1, nchunk)):
            start_in(ch)
        for ch in range(nchunk):
            la = ch + DEPTH - 1
            if la < nchunk:
                if la >= DEPTH:
                    wait_out(la - DEPTH)
                start_in(la)
            wait_in(ch)
            start_out(ch)
        for ch in range(max(0, nchunk - DEPTH), nchunk):
            wait_out(ch)

    return pl.pallas_call(
        body,
        in_specs=[pl.BlockSpec(memory_space=pltpu.MemorySpace.HBM)],
        out_specs=pl.BlockSpec(memory_space=pltpu.MemorySpace.HBM),
        out_shape=jax.ShapeDtypeStruct(flat.shape, flat.dtype),
        scratch_shapes=[
            pltpu.VMEM((DEPTH, CH, H, W), flat.dtype),
            pltpu.SemaphoreType.DMA((DEPTH,)),
            pltpu.SemaphoreType.DMA((DEPTH,)),
        ],
    )(flat)


def kernel(frames):
    C, T, H, W = frames.shape
    n = T // _ALPHA
    idx = _linspace_trunc_idx(T)
    assert all(i * (T - 1) // (n - 1) == v for i, v in enumerate(idx)), idx

    flat = frames.reshape(C * T, H, W)
    slow = _sc_gather(flat, C, T, H, W, n)
    fast = _tc_copy(flat)
    return (slow.reshape(C, n, H, W), fast.reshape(C, T, H, W))


# speedup vs baseline: 1.0071x; 1.0071x over previous
import functools

import jax
import jax.numpy as jnp
import numpy as np
from jax import lax
from jax.experimental import pallas as pl
from jax.experimental.pallas import tpu as pltpu
from jax.experimental.pallas import tpu_sc as plsc

_ALPHA = 4


def _linspace_trunc_idx(t: int) -> tuple:
    with jax.ensure_compile_time_eval():
        v = jnp.linspace(0.0, t - 1, t // _ALPHA).astype(jnp.int32)
    return tuple(int(i) for i in np.asarray(v))


def _sc_gather(flat, C, T, H, W, n):
    nslab = C * n
    ROWS = 64
    per_slab = H // ROWS
    nunit = nslab * per_slab
    mesh = plsc.VectorSubcoreMesh(core_axis_name="c", subcore_axis_name="s")
    info = plsc.get_sparse_core_info()
    nworker = info.num_cores * info.num_subcores
    per_worker = nunit // nworker
    assert nunit % nworker == 0

    @functools.partial(
        pl.kernel,
        mesh=mesh,
        out_type=jax.ShapeDtypeStruct((nslab, H, W), flat.dtype),
        scratch_types=[
            pltpu.VMEM((per_worker, ROWS, W), flat.dtype),
            pltpu.SemaphoreType.DMA((per_worker,)),
            pltpu.SemaphoreType.DMA((per_worker,)),
        ],
    )
    def sc_kernel(x_hbm, o_hbm, buf, in_sem, out_sem):
        wid = lax.axis_index("s") * info.num_cores + lax.axis_index("c")

        def slices(k):
            u = wid * per_worker + k
            j = u // per_slab
            p = u % per_slab
            c = j // n
            t = j % n
            src = c * T + t * (T - 1) // (n - 1)
            row0 = p * ROWS
            return (
                x_hbm.at[src, pl.ds(row0, ROWS), :],
                o_hbm.at[j, pl.ds(row0, ROWS), :],
            )

        ins = []
        for k in range(per_worker):
            src_slice, _ = slices(k)
            ins.append(pltpu.async_copy(src_slice, buf.at[k], in_sem.at[k]))
        outs = []
        for k in range(per_worker):
            _, dst_slice = slices(k)
            ins[k].wait()
            outs.append(pltpu.async_copy(buf.at[k], dst_slice, out_sem.at[k]))
        for cp in outs:
            cp.wait()

    return sc_kernel(flat)


def _tc_copy(flat):
    N, H, W = flat.shape
    CH = 16
    nchunk = N // CH
    DEPTH = 3

    def body(src, dst, buf, in_sem, out_sem):
        def start_in(ch):
            b = ch % DEPTH
            pltpu.make_async_copy(
                src.at[pl.ds(ch * CH, CH)], buf.at[b], in_sem.at[b]
            ).start()

        def wait_in(ch):
            b = ch % DEPTH
            pltpu.make_async_copy(
                src.at[pl.ds(ch * CH, CH)], buf.at[b], in_sem.at[b]
            ).wait()

        def start_out(ch):
            b = ch % DEPTH
            pltpu.make_async_copy(
                buf.at[b], dst.at[pl.ds(ch * CH, CH)], out_sem.at[b]
            ).start()

        def wait_out(ch):
            b = ch % DEPTH
            pltpu.make_async_copy(
                buf.at[b], dst.at[pl.ds(ch * CH, CH)], out_sem.at[b]
            ).wait()

        for ch in range(min(DEPTH - 1, nchunk)):
            start_in(ch)
        for ch in range(nchunk):
            la = ch + DEPTH - 1
            if la < nchunk:
                if la >= DEPTH:
                    wait_out(la - DEPTH)
                start_in(la)
            wait_in(ch)
            start_out(ch)
        for ch in range(max(0, nchunk - DEPTH), nchunk):
            wait_out(ch)

    return pl.pallas_call(
        body,
        in_specs=[pl.BlockSpec(memory_space=pltpu.MemorySpace.HBM)],
        out_specs=pl.BlockSpec(memory_space=pltpu.MemorySpace.HBM),
        out_shape=jax.ShapeDtypeStruct(flat.shape, flat.dtype),
        scratch_shapes=[
            pltpu.VMEM((DEPTH, CH, H, W), flat.dtype),
            pltpu.SemaphoreType.DMA((DEPTH,)),
            pltpu.SemaphoreType.DMA((DEPTH,)),
        ],
    )(flat)


def kernel(frames):
    C, T, H, W = frames.shape
    n = T // _ALPHA
    idx = _linspace_trunc_idx(T)
    assert all(i * (T - 1) // (n - 1) == v for i, v in enumerate(idx)), idx

    flat = frames.reshape(C * T, H, W)
    fast = _tc_copy(flat)
    slow = _sc_gather(flat, C, T, H, W, n)
    return (slow.reshape(C, n, H, W), fast.reshape(C, T, H, W))


# speedup vs baseline: 1.4399x; 1.4298x over previous
import jax
import jax.numpy as jnp
import numpy as np
from jax.experimental import pallas as pl
from jax.experimental.pallas import tpu as pltpu

_ALPHA = 4


def _linspace_trunc_idx(t: int) -> tuple:
    with jax.ensure_compile_time_eval():
        v = jnp.linspace(0.0, t - 1, t // _ALPHA).astype(jnp.int32)
    return tuple(int(i) for i in np.asarray(v))


def kernel(frames):
    C, T, H, W = frames.shape
    n = T // _ALPHA
    idx = _linspace_trunc_idx(T)

    CH = 8
    nchunk = (C * T) // CH
    DEPTH = 4
    sel = {ch: [] for ch in range(nchunk)}
    for c in range(C):
        for k, s in enumerate(idx):
            g = c * T + s
            sel[g // CH].append((g % CH, c * n + k))

    def body(src, slow, fast, inbuf, slowbuf, in_sem, out_sem, slow_sem):
        def start_in(ch):
            b = ch % DEPTH
            pltpu.make_async_copy(
                src.at[pl.ds(ch * CH, CH)], inbuf.at[b], in_sem.at[b]
            ).start()

        def wait_in(ch):
            b = ch % DEPTH
            pltpu.make_async_copy(
                src.at[pl.ds(ch * CH, CH)], inbuf.at[b], in_sem.at[b]
            ).wait()

        def start_out(ch):
            b = ch % DEPTH
            pltpu.make_async_copy(
                inbuf.at[b], fast.at[pl.ds(ch * CH, CH)], out_sem.at[b]
            ).start()

        def wait_out(ch):
            b = ch % DEPTH
            pltpu.make_async_copy(
                inbuf.at[b], fast.at[pl.ds(ch * CH, CH)], out_sem.at[b]
            ).wait()

        def slow_flush(c):
            return pltpu.make_async_copy(
                slowbuf.at[pl.ds(c * n, n)],
                slow.at[pl.ds(c * n, n)],
                slow_sem.at[c],
            )

        flush_after = {(c * T + T - 1) // CH: c for c in range(C)}

        for ch in range(min(DEPTH - 1, nchunk)):
            start_in(ch)
        for ch in range(nchunk):
            la = ch + DEPTH - 1
            if la < nchunk:
                if la >= DEPTH:
                    wait_out(la - DEPTH)
                start_in(la)
            wait_in(ch)
            start_out(ch)
            for off, j in sel[ch]:
                slowbuf[j] = inbuf[ch % DEPTH, off]
            if ch in flush_after:
                slow_flush(flush_after[ch]).start()
        for ch in range(max(0, nchunk - DEPTH), nchunk):
            wait_out(ch)
        for c in range(C):
            slow_flush(c).wait()

    flat = frames.reshape(C * T, H, W)
    slow, fast = pl.pallas_call(
        body,
        in_specs=[pl.BlockSpec(memory_space=pltpu.MemorySpace.HBM)],
        out_specs=(
            pl.BlockSpec(memory_space=pltpu.MemorySpace.HBM),
            pl.BlockSpec(memory_space=pltpu.MemorySpace.HBM),
        ),
        out_shape=(
            jax.ShapeDtypeStruct((C * n, H, W), frames.dtype),
            jax.ShapeDtypeStruct((C * T, H, W), frames.dtype),
        ),
        scratch_shapes=[
            pltpu.VMEM((DEPTH, CH, H, W), frames.dtype),
            pltpu.VMEM((C * n, H, W), frames.dtype),
            pltpu.SemaphoreType.DMA((DEPTH,)),
            pltpu.SemaphoreType.DMA((DEPTH,)),
            pltpu.SemaphoreType.DMA((C,)),
        ],
    )(flat)
    return (slow.reshape(C, n, H, W), fast.reshape(C, T, H, W))


# speedup vs baseline: 1.5281x; 1.0613x over previous
import jax
import jax.numpy as jnp
import numpy as np
from jax.experimental import pallas as pl
from jax.experimental.pallas import tpu as pltpu

_ALPHA = 4


def _linspace_trunc_idx(t: int) -> tuple:
    with jax.ensure_compile_time_eval():
        v = jnp.linspace(0.0, t - 1, t // _ALPHA).astype(jnp.int32)
    return tuple(int(i) for i in np.asarray(v))


def kernel(frames):
    C, T, H, W = frames.shape
    n = T // _ALPHA
    idx = _linspace_trunc_idx(T)

    CH = 16
    nchunk = (C * T) // CH
    DEPTH = 3
    sel = {ch: [] for ch in range(nchunk)}
    for c in range(C):
        for k, s in enumerate(idx):
            g = c * T + s
            sel[g // CH].append((g % CH, c * n + k))

    def body(src, slow, fast, inbuf, slowbuf, in_sem, out_sem, slow_sem):
        def start_in(ch):
            b = ch % DEPTH
            pltpu.make_async_copy(
                src.at[pl.ds(ch * CH, CH)], inbuf.at[b], in_sem.at[b]
            ).start()

        def wait_in(ch):
            b = ch % DEPTH
            pltpu.make_async_copy(
                src.at[pl.ds(ch * CH, CH)], inbuf.at[b], in_sem.at[b]
            ).wait()

        def start_out(ch):
            b = ch % DEPTH
            pltpu.make_async_copy(
                inbuf.at[b], fast.at[pl.ds(ch * CH, CH)], out_sem.at[b]
            ).start()

        def wait_out(ch):
            b = ch % DEPTH
            pltpu.make_async_copy(
                inbuf.at[b], fast.at[pl.ds(ch * CH, CH)], out_sem.at[b]
            ).wait()

        def slow_flush(c):
            return pltpu.make_async_copy(
                slowbuf.at[pl.ds(c * n, n)],
                slow.at[pl.ds(c * n, n)],
                slow_sem.at[c],
            )

        flush_after = {(c * T + T - 1) // CH: c for c in range(C)}

        for ch in range(min(DEPTH - 1, nchunk)):
            start_in(ch)
        for ch in range(nchunk):
            la = ch + DEPTH - 1
            if la < nchunk:
                if la >= DEPTH:
                    wait_out(la - DEPTH)
                start_in(la)
            wait_in(ch)
            start_out(ch)
            for off, j in sel[ch]:
                slowbuf[j] = inbuf[ch % DEPTH, off]
            if ch in flush_after:
                slow_flush(flush_after[ch]).start()
        for ch in range(max(0, nchunk - DEPTH), nchunk):
            wait_out(ch)
        for c in range(C):
            slow_flush(c).wait()

    flat = frames.reshape(C * T, H, W)
    slow, fast = pl.pallas_call(
        body,
        in_specs=[pl.BlockSpec(memory_space=pltpu.MemorySpace.HBM)],
        out_specs=(
            pl.BlockSpec(memory_space=pltpu.MemorySpace.HBM),
            pl.BlockSpec(memory_space=pltpu.MemorySpace.HBM),
        ),
        out_shape=(
            jax.ShapeDtypeStruct((C * n, H, W), frames.dtype),
            jax.ShapeDtypeStruct((C * T, H, W), frames.dtype),
        ),
        scratch_shapes=[
            pltpu.VMEM((DEPTH, CH, H, W), frames.dtype),
            pltpu.VMEM((C * n, H, W), frames.dtype),
            pltpu.SemaphoreType.DMA((DEPTH,)),
            pltpu.SemaphoreType.DMA((DEPTH,)),
            pltpu.SemaphoreType.DMA((C,)),
        ],
    )(flat)
    return (slow.reshape(C, n, H, W), fast.reshape(C, T, H, W))


# speedup vs baseline: 1.6383x; 1.0721x over previous
import jax
import jax.numpy as jnp
import numpy as np
from jax.experimental import pallas as pl
from jax.experimental.pallas import tpu as pltpu

_ALPHA = 4


def _linspace_trunc_idx(t: int) -> tuple:
    with jax.ensure_compile_time_eval():
        v = jnp.linspace(0.0, t - 1, t // _ALPHA).astype(jnp.int32)
    return tuple(int(i) for i in np.asarray(v))


def kernel(frames):
    C, T, H, W = frames.shape
    n = T // _ALPHA
    idx = _linspace_trunc_idx(T)

    CH = 32
    nchunk = (C * T) // CH
    DEPTH = 3
    sel = {ch: [] for ch in range(nchunk)}
    for c in range(C):
        for k, s in enumerate(idx):
            g = c * T + s
            sel[g // CH].append((g % CH, c * n + k))

    def body(src, slow, fast, inbuf, slowbuf, in_sem, out_sem, slow_sem):
        def start_in(ch):
            b = ch % DEPTH
            pltpu.make_async_copy(
                src.at[pl.ds(ch * CH, CH)], inbuf.at[b], in_sem.at[b]
            ).start()

        def wait_in(ch):
            b = ch % DEPTH
            pltpu.make_async_copy(
                src.at[pl.ds(ch * CH, CH)], inbuf.at[b], in_sem.at[b]
            ).wait()

        def start_out(ch):
            b = ch % DEPTH
            pltpu.make_async_copy(
                inbuf.at[b], fast.at[pl.ds(ch * CH, CH)], out_sem.at[b]
            ).start()

        def wait_out(ch):
            b = ch % DEPTH
            pltpu.make_async_copy(
                inbuf.at[b], fast.at[pl.ds(ch * CH, CH)], out_sem.at[b]
            ).wait()

        def slow_flush(c):
            return pltpu.make_async_copy(
                slowbuf.at[pl.ds(c * n, n)],
                slow.at[pl.ds(c * n, n)],
                slow_sem.at[c],
            )

        flush_after = {(c * T + T - 1) // CH: c for c in range(C)}

        for ch in range(min(DEPTH - 1, nchunk)):
            start_in(ch)
        for ch in range(nchunk):
            la = ch + DEPTH - 1
            if la < nchunk:
                if la >= DEPTH:
                    wait_out(la - DEPTH)
                start_in(la)
            wait_in(ch)
            start_out(ch)
            for off, j in sel[ch]:
                slowbuf[j] = inbuf[ch % DEPTH, off]
            if ch in flush_after:
                slow_flush(flush_after[ch]).start()
        for ch in range(max(0, nchunk - DEPTH), nchunk):
            wait_out(ch)
        for c in range(C):
            slow_flush(c).wait()

    flat = frames.reshape(C * T, H, W)
    slow, fast = pl.pallas_call(
        body,
        in_specs=[pl.BlockSpec(memory_space=pltpu.MemorySpace.HBM)],
        out_specs=(
            pl.BlockSpec(memory_space=pltpu.MemorySpace.HBM),
            pl.BlockSpec(memory_space=pltpu.MemorySpace.HBM),
        ),
        out_shape=(
            jax.ShapeDtypeStruct((C * n, H, W), frames.dtype),
            jax.ShapeDtypeStruct((C * T, H, W), frames.dtype),
        ),
        scratch_shapes=[
            pltpu.VMEM((DEPTH, CH, H, W), frames.dtype),
            pltpu.VMEM((C * n, H, W), frames.dtype),
            pltpu.SemaphoreType.DMA((DEPTH,)),
            pltpu.SemaphoreType.DMA((DEPTH,)),
            pltpu.SemaphoreType.DMA((C,)),
        ],
    )(flat)
    return (slow.reshape(C, n, H, W), fast.reshape(C, T, H, W))


# speedup vs baseline: 1.6468x; 1.0052x over previous
import jax
import jax.numpy as jnp
import numpy as np
from jax.experimental import pallas as pl
from jax.experimental.pallas import tpu as pltpu

_ALPHA = 4


def _linspace_trunc_idx(t: int) -> tuple:
    with jax.ensure_compile_time_eval():
        v = jnp.linspace(0.0, t - 1, t // _ALPHA).astype(jnp.int32)
    return tuple(int(i) for i in np.asarray(v))


def kernel(frames):
    C, T, H, W = frames.shape
    n = T // _ALPHA
    idx = _linspace_trunc_idx(T)

    CH = 64
    nchunk = (C * T) // CH
    DEPTH = 2
    sel = {ch: [] for ch in range(nchunk)}
    for c in range(C):
        for k, s in enumerate(idx):
            g = c * T + s
            sel[g // CH].append((g % CH, c * n + k))

    def body(src, slow, fast, inbuf, slowbuf, in_sem, out_sem, slow_sem):
        def start_in(ch):
            b = ch % DEPTH
            pltpu.make_async_copy(
                src.at[pl.ds(ch * CH, CH)], inbuf.at[b], in_sem.at[b]
            ).start()

        def wait_in(ch):
            b = ch % DEPTH
            pltpu.make_async_copy(
                src.at[pl.ds(ch * CH, CH)], inbuf.at[b], in_sem.at[b]
            ).wait()

        def start_out(ch):
            b = ch % DEPTH
            pltpu.make_async_copy(
                inbuf.at[b], fast.at[pl.ds(ch * CH, CH)], out_sem.at[b]
            ).start()

        def wait_out(ch):
            b = ch % DEPTH
            pltpu.make_async_copy(
                inbuf.at[b], fast.at[pl.ds(ch * CH, CH)], out_sem.at[b]
            ).wait()

        def slow_flush(c):
            return pltpu.make_async_copy(
                slowbuf.at[pl.ds(c * n, n)],
                slow.at[pl.ds(c * n, n)],
                slow_sem.at[c],
            )

        flush_after = {(c * T + T - 1) // CH: c for c in range(C)}

        for ch in range(min(DEPTH - 1, nchunk)):
            start_in(ch)
        for ch in range(nchunk):
            la = ch + DEPTH - 1
            if la < nchunk:
                if la >= DEPTH:
                    wait_out(la - DEPTH)
                start_in(la)
            wait_in(ch)
            start_out(ch)
            for off, j in sel[ch]:
                slowbuf[j] = inbuf[ch % DEPTH, off]
            if ch in flush_after:
                slow_flush(flush_after[ch]).start()
        for ch in range(max(0, nchunk - DEPTH), nchunk):
            wait_out(ch)
        for c in range(C):
            slow_flush(c).wait()

    flat = frames.reshape(C * T, H, W)
    slow, fast = pl.pallas_call(
        body,
        in_specs=[pl.BlockSpec(memory_space=pltpu.MemorySpace.HBM)],
        out_specs=(
            pl.BlockSpec(memory_space=pltpu.MemorySpace.HBM),
            pl.BlockSpec(memory_space=pltpu.MemorySpace.HBM),
        ),
        out_shape=(
            jax.ShapeDtypeStruct((C * n, H, W), frames.dtype),
            jax.ShapeDtypeStruct((C * T, H, W), frames.dtype),
        ),
        scratch_shapes=[
            pltpu.VMEM((DEPTH, CH, H, W), frames.dtype),
            pltpu.VMEM((C * n, H, W), frames.dtype),
            pltpu.SemaphoreType.DMA((DEPTH,)),
            pltpu.SemaphoreType.DMA((DEPTH,)),
            pltpu.SemaphoreType.DMA((C,)),
        ],
    )(flat)
    return (slow.reshape(C, n, H, W), fast.reshape(C, T, H, W))
